# baseline (speedup 1.0000x reference)
import functools
import jax
import jax.numpy as jnp
from jax import lax
from jax.experimental import pallas as pl
from jax.experimental.pallas import tpu as pltpu
from jax.experimental.pallas import tpu_sc as plsc

N = 10000
E = 160000
D = 256
HALF = 128
NC = 2
NS = 16
NW = NC * NS
UNIT = 128
E_PAD = 163840
UNITS = E_PAD // UNIT
UPW = UNITS // NW
UPT = UNITS // NS
N_PAD = 10240
NPT = N_PAD // NS
BT = 400
NB = N // BT
NBUF = 2
NSPLIT = 1
UPP = UPT // 2

def _fill_zero_2d(buf, nrows):
    zero16 = jnp.zeros((16,), jnp.float32)

    @pl.loop(0, nrows)
    def _z(r):
        for k in range(HALF // 16):
            buf[r, pl.ds(k * 16, 16)] = zero16


@functools.cache
def _make_deg_kernel():
    mesh = plsc.VectorSubcoreMesh(
        core_axis_name="c", subcore_axis_name="s", num_cores=NC, num_subcores=NS
    )
    return functools.partial(
        pl.kernel,
        out_type=(
            jax.ShapeDtypeStruct((NC, N_PAD), jnp.float32),
            jax.ShapeDtypeStruct((NC, N_PAD), jnp.float32),
        ),
        mesh=mesh,
        scratch_types=(
            pltpu.VMEM((UNIT,), jnp.int32),
            pltpu.VMEM((UNIT,), jnp.int32),
            pltpu.VMEM((UNIT,), jnp.float32),
            pltpu.VMEM((NPT,), jnp.float32),
            pltpu.VMEM_SHARED((N_PAD,), jnp.float32),
            pltpu.VMEM_SHARED((N_PAD,), jnp.float32),
        ),
    )(_deg_body)


def _deg_body(srcR, dstR, wR, dego_out, degi_out, sv, dv, wv, stage, dego, degi):
    c = lax.axis_index("c")
    s = lax.axis_index("s")

    zero16 = jnp.zeros((16,), jnp.float32)

    @pl.loop(0, NPT // 16)
    def _z(r):
        stage[pl.ds(r * 16, 16)] = zero16

    pltpu.sync_copy(stage, dego.at[pl.ds(s * NPT, NPT)])
    pltpu.sync_copy(stage, degi.at[pl.ds(s * NPT, NPT)])
    plsc.subcore_barrier()

    base_u = (c * NS + s) * UPW

    @pl.loop(0, UPW)
    def _unit(u):
        row = base_u + u
        pltpu.sync_copy(srcR.at[row], sv)
        pltpu.sync_copy(dstR.at[row], dv)
        pltpu.sync_copy(wR.at[row], wv)
        pltpu.sync_copy(wv, dego.at[sv], add=True)
        pltpu.sync_copy(wv, degi.at[dv], add=True)

    plsc.subcore_barrier()

    pltpu.sync_copy(dego.at[pl.ds(s * NPT, NPT)], stage)
    pltpu.sync_copy(stage, dego_out.at[c, pl.ds(s * NPT, NPT)])
    pltpu.sync_copy(degi.at[pl.ds(s * NPT, NPT)], stage)
    pltpu.sync_copy(stage, degi_out.at[c, pl.ds(s * NPT, NPT)])


@functools.cache
def _make_msg_kernel():
    mesh = plsc.VectorSubcoreMesh(
        core_axis_name="c", subcore_axis_name="s", num_cores=NC, num_subcores=NS
    )
    return functools.partial(
        pl.kernel,
        out_type=jax.ShapeDtypeStruct((NC, N_PAD, HALF), jnp.float32),
        mesh=mesh,
        scratch_types=(
            pltpu.VMEM((UPP, UNIT), jnp.int32),
            pltpu.VMEM((UPP, UNIT), jnp.int32),
            pltpu.VMEM((UPP, UNIT), jnp.float32),
            tuple(pltpu.VMEM((UNIT, HALF), jnp.float32) for _ in range(NBUF)),
            tuple(
                tuple(pltpu.SemaphoreType.DMA for _ in range(NSPLIT))
                for _ in range(NBUF)
            ),
            tuple(pltpu.SemaphoreType.DMA for _ in range(NBUF)),
            pltpu.VMEM_SHARED((N_PAD, HALF), jnp.float32),
        ),
    )(_msg_body)


def _msg_body(yflat, srcR, dstR, wR, out, src_st, dst_st, w_st, rows, gsem, ssem, acc):
    c = lax.axis_index("c")
    s = lax.axis_index("s")
    coff = c * N
    SP = UNIT // NSPLIT

    _fill_zero_2d(rows[0], UNIT)
    for p in range(NPT // UNIT):
        pltpu.sync_copy(rows[0], acc.at[pl.ds(s * NPT + p * UNIT, UNIT)])
    plsc.subcore_barrier()

    for half in range(2):
        base = s * UPT + half * UPP
        pltpu.sync_copy(srcR.at[pl.ds(base, UPP)], src_st)
        pltpu.sync_copy(dstR.at[pl.ds(base, UPP)], dst_st)
        pltpu.sync_copy(wR.at[pl.ds(base, UPP)], w_st)

        @pl.loop(0, UPP)
        def _gi(u):
            for k in range(UNIT // 16):
                src_st[u, pl.ds(k * 16, 16)] = src_st[u, pl.ds(k * 16, 16)] + coff

        def _gather(u, b):
            urow = jnp.minimum(u, UPP - 1)
            for q in range(NSPLIT):
                pltpu.async_copy(
                    yflat.at[src_st.at[urow, pl.ds(q * SP, SP)]],
                    rows[b].at[pl.ds(q * SP, SP)],
                    gsem[b][q],
                )

        def _gwait(b):
            for q in range(NSPLIT):
                pltpu.make_async_copy(
                    yflat.at[src_st.at[0, pl.ds(0, SP)]],
                    rows[b].at[pl.ds(0, SP)],
                    gsem[b][q],
                ).wait()

        def _scale(u, b):
            @pl.loop(0, UNIT // 16)
            def _jb(jb):
                wvec = w_st[u, pl.ds(jb * 16, 16)]
                for l in range(16):
                    wl = wvec[l]
                    r = jb * 16 + l
                    for k in range(HALF // 16):
                        rows[b][r, pl.ds(k * 16, 16)] = (
                            rows[b][r, pl.ds(k * 16, 16)] * wl
                        )

        for b in range(NBUF):
            _gather(b, b)

        @pl.loop(0, UPP, step=NBUF)
        def _round(i):
            for b in range(NBUF):
                u = i + b
                _gwait(b)
                _scale(u, b)
                pltpu.async_copy(rows[b], acc.at[dst_st.at[u]], ssem[b], add=True)
            for b in range(NBUF):
                pltpu.make_async_copy(rows[b], acc.at[dst_st.at[0]], ssem[b]).wait()
                _gather(i + NBUF + b, b)

        for b in range(NBUF):
            _gwait(b)

    plsc.subcore_barrier()

    for p in range(NPT // UNIT):
        pltpu.sync_copy(acc.at[pl.ds(s * NPT + p * UNIT, UNIT)], rows[0])
        pltpu.sync_copy(rows[0], out.at[c, pl.ds(s * NPT + p * UNIT, UNIT)])


def _mm1_body(dego_ref, degi_ref, h_ref, w1_ref, y_ref, rin_ref, rout_ref):
    d_out = dego_ref[0, :, 0] + dego_ref[1, :, 0]
    d_in = degi_ref[0, :, 0] + degi_ref[1, :, 0]
    r_out = lax.rsqrt(jnp.maximum(d_out, 1e-12))
    r_in = lax.rsqrt(jnp.maximum(d_in, 1e-12))
    rout_ref[...] = r_out[:, None]
    rin_ref[...] = r_in[:, None]
    x = h_ref[...] * r_out[:, None]
    y_ref[0] = jnp.dot(x, w1_ref[...], preferred_element_type=jnp.float32)


_MM1_ARGS = dict(
    grid=(NC, NB),
    in_specs=[
        pl.BlockSpec((NC, BT, 1), lambda c, i: (0, i, 0)),
        pl.BlockSpec((NC, BT, 1), lambda c, i: (0, i, 0)),
        pl.BlockSpec((BT, D), lambda c, i: (i, 0)),
        pl.BlockSpec((D, HALF), lambda c, i: (0, c)),
    ],
    out_specs=[
        pl.BlockSpec((1, BT, HALF), lambda c, i: (c, i, 0)),
        pl.BlockSpec((BT, 1), lambda c, i: (i, 0)),
        pl.BlockSpec((BT, 1), lambda c, i: (i, 0)),
    ],
    out_shape=[
        jax.ShapeDtypeStruct((NC, N, HALF), jnp.float32),
        jax.ShapeDtypeStruct((N, 1), jnp.float32),
        jax.ShapeDtypeStruct((N, 1), jnp.float32),
    ],
)
_mm1 = pl.pallas_call(_mm1_body, **_MM1_ARGS)


def _mm2_body(a_ref, rin_ref, rout_ref, b1_ref, w2_ref, y2_ref):
    rin = rin_ref[...]
    rout = rout_ref[...]
    h0 = jax.nn.relu(a_ref[0] * rin + b1_ref[0][None, :]) * rout
    h1 = jax.nn.relu(a_ref[1] * rin + b1_ref[1][None, :]) * rout
    y2_ref[0] = (
        jnp.dot(h0, w2_ref[0], preferred_element_type=jnp.float32)
        + jnp.dot(h1, w2_ref[1], preferred_element_type=jnp.float32)
    )


_MM2_ARGS = dict(
    grid=(NC, NB),
    in_specs=[
        pl.BlockSpec((NC, BT, HALF), lambda c, i: (0, i, 0)),
        pl.BlockSpec((BT, 1), lambda c, i: (i, 0)),
        pl.BlockSpec((BT, 1), lambda c, i: (i, 0)),
        pl.BlockSpec((NC, HALF), lambda c, i: (0, 0)),
        pl.BlockSpec((NC, HALF, HALF), lambda c, i: (0, 0, c)),
    ],
    out_specs=pl.BlockSpec((1, BT, HALF), lambda c, i: (c, i, 0)),
    out_shape=jax.ShapeDtypeStruct((NC, N, HALF), jnp.float32),
)
_mm2 = pl.pallas_call(_mm2_body, **_MM2_ARGS)


def _fin_body(a_ref, rin_ref, b2_ref, out_ref):
    rin = rin_ref[...]
    b2 = b2_ref[...]
    lo = a_ref[0] * rin + b2[:HALF][None, :]
    hi = a_ref[1] * rin + b2[HALF:][None, :]
    out_ref[...] = jnp.concatenate([lo, hi], axis=1)


_FIN_ARGS = dict(
    grid=(NB,),
    in_specs=[
        pl.BlockSpec((NC, BT, HALF), lambda i: (0, i, 0)),
        pl.BlockSpec((BT, 1), lambda i: (i, 0)),
        pl.BlockSpec((D,), lambda i: (0,)),
    ],
    out_specs=pl.BlockSpec((BT, D), lambda i: (i, 0)),
    out_shape=jax.ShapeDtypeStruct((N, D), jnp.float32),
)
_fin = pl.pallas_call(_fin_body, **_FIN_ARGS)


def kernel(h, edge_index, w, W1, b1, W2, b2):
    src = edge_index[0].astype(jnp.int32)
    dst = edge_index[1].astype(jnp.int32)
    w = w.astype(jnp.float32)
    pad = E_PAD - E
    srcR = jnp.concatenate([src, jnp.zeros((pad,), jnp.int32)]).reshape(UNITS, UNIT)
    dstR = jnp.concatenate([dst, jnp.zeros((pad,), jnp.int32)]).reshape(UNITS, UNIT)
    wR = jnp.concatenate([w, jnp.zeros((pad,), jnp.float32)]).reshape(UNITS, UNIT)

    deg_kernel = _make_deg_kernel()
    msg_kernel = _make_msg_kernel()
    dego, degi = deg_kernel(srcR, dstR, wR)
    y1, rin, rout = _mm1(
        dego.reshape(NC, N_PAD, 1), degi.reshape(NC, N_PAD, 1), h, W1
    )
    a1 = msg_kernel(y1.reshape(NC * N, HALF), srcR, dstR, wR)
    y2 = _mm2(a1, rin, rout, b1.reshape(NC, HALF), W2.reshape(NC, HALF, D))
    a2 = msg_kernel(y2.reshape(NC * N, HALF), srcR, dstR, wR)
    return _fin(a2, rin, b2)


# speedup vs baseline: 1.0584x; 1.0584x over previous
import functools
import jax
import jax.numpy as jnp
from jax import lax
from jax.experimental import pallas as pl
from jax.experimental.pallas import tpu as pltpu
from jax.experimental.pallas import tpu_sc as plsc

N = 10000
E = 160000
D = 256
HALF = 128
NC = 2
NS = 16
NW = NC * NS
UNIT = 128
E_PAD = 163840
UNITS = E_PAD // UNIT
UPW = UNITS // NW
UPT = UNITS // NS
N_PAD = 10240
NPT = N_PAD // NS
BT = 512
NB = N_PAD // BT
NBUF = 2
NSPLIT = 1
UPP = UPT // 2

def _fill_zero_2d(buf, nrows):
    zero16 = jnp.zeros((16,), jnp.float32)

    @pl.loop(0, nrows)
    def _z(r):
        for k in range(HALF // 16):
            buf[r, pl.ds(k * 16, 16)] = zero16


@functools.cache
def _make_deg_kernel():
    mesh = plsc.VectorSubcoreMesh(
        core_axis_name="c", subcore_axis_name="s", num_cores=NC, num_subcores=NS
    )
    return functools.partial(
        pl.kernel,
        out_type=(
            jax.ShapeDtypeStruct((NC, N_PAD), jnp.float32),
            jax.ShapeDtypeStruct((NC, N_PAD), jnp.float32),
        ),
        mesh=mesh,
        scratch_types=(
            pltpu.VMEM((UNIT,), jnp.int32),
            pltpu.VMEM((UNIT,), jnp.int32),
            pltpu.VMEM((UNIT,), jnp.float32),
            pltpu.VMEM((NPT,), jnp.float32),
            pltpu.VMEM_SHARED((N_PAD,), jnp.float32),
            pltpu.VMEM_SHARED((N_PAD,), jnp.float32),
        ),
    )(_deg_body)


def _deg_body(srcR, dstR, wR, dego_out, degi_out, sv, dv, wv, stage, dego, degi):
    c = lax.axis_index("c")
    s = lax.axis_index("s")

    zero16 = jnp.zeros((16,), jnp.float32)

    @pl.loop(0, NPT // 16)
    def _z(r):
        stage[pl.ds(r * 16, 16)] = zero16

    pltpu.sync_copy(stage, dego.at[pl.ds(s * NPT, NPT)])
    pltpu.sync_copy(stage, degi.at[pl.ds(s * NPT, NPT)])
    plsc.subcore_barrier()

    base_u = (c * NS + s) * UPW

    @pl.loop(0, UPW)
    def _unit(u):
        row = base_u + u
        pltpu.sync_copy(srcR.at[row], sv)
        pltpu.sync_copy(dstR.at[row], dv)
        pltpu.sync_copy(wR.at[row], wv)
        pltpu.sync_copy(wv, dego.at[sv], add=True)
        pltpu.sync_copy(wv, degi.at[dv], add=True)

    plsc.subcore_barrier()

    pltpu.sync_copy(dego.at[pl.ds(s * NPT, NPT)], stage)
    pltpu.sync_copy(stage, dego_out.at[c, pl.ds(s * NPT, NPT)])
    pltpu.sync_copy(degi.at[pl.ds(s * NPT, NPT)], stage)
    pltpu.sync_copy(stage, degi_out.at[c, pl.ds(s * NPT, NPT)])


@functools.cache
def _make_msg_kernel():
    mesh = plsc.VectorSubcoreMesh(
        core_axis_name="c", subcore_axis_name="s", num_cores=NC, num_subcores=NS
    )
    return functools.partial(
        pl.kernel,
        out_type=jax.ShapeDtypeStruct((NC, N_PAD, HALF), jnp.float32),
        mesh=mesh,
        scratch_types=(
            pltpu.VMEM((UPP, UNIT), jnp.int32),
            pltpu.VMEM((UPP, UNIT), jnp.int32),
            pltpu.VMEM((UPP, UNIT), jnp.float32),
            tuple(pltpu.VMEM((UNIT, HALF), jnp.float32) for _ in range(NBUF)),
            tuple(
                tuple(pltpu.SemaphoreType.DMA for _ in range(NSPLIT))
                for _ in range(NBUF)
            ),
            tuple(pltpu.SemaphoreType.DMA for _ in range(NBUF)),
            pltpu.VMEM_SHARED((N_PAD, HALF), jnp.float32),
        ),
    )(_msg_body)


def _msg_body(yflat, srcR, dstR, wR, out, src_st, dst_st, w_st, rows, gsem, ssem, acc):
    c = lax.axis_index("c")
    s = lax.axis_index("s")
    coff = c * N_PAD
    SP = UNIT // NSPLIT

    _fill_zero_2d(rows[0], UNIT)
    for p in range(NPT // UNIT):
        pltpu.sync_copy(rows[0], acc.at[pl.ds(s * NPT + p * UNIT, UNIT)])
    plsc.subcore_barrier()

    for half in range(2):
        base = s * UPT + half * UPP
        pltpu.sync_copy(srcR.at[pl.ds(base, UPP)], src_st)
        pltpu.sync_copy(dstR.at[pl.ds(base, UPP)], dst_st)
        pltpu.sync_copy(wR.at[pl.ds(base, UPP)], w_st)

        @pl.loop(0, UPP)
        def _gi(u):
            for k in range(UNIT // 16):
                src_st[u, pl.ds(k * 16, 16)] = src_st[u, pl.ds(k * 16, 16)] + coff

        def _gather(u, b):
            urow = jnp.minimum(u, UPP - 1)
            for q in range(NSPLIT):
                pltpu.async_copy(
                    yflat.at[src_st.at[urow, pl.ds(q * SP, SP)]],
                    rows[b].at[pl.ds(q * SP, SP)],
                    gsem[b][q],
                )

        def _gwait(b):
            for q in range(NSPLIT):
                pltpu.make_async_copy(
                    yflat.at[src_st.at[0, pl.ds(0, SP)]],
                    rows[b].at[pl.ds(0, SP)],
                    gsem[b][q],
                ).wait()

        def _scale(u, b):
            @pl.loop(0, UNIT // 16)
            def _jb(jb):
                wvec = w_st[u, pl.ds(jb * 16, 16)]
                for l in range(16):
                    wl = wvec[l]
                    r = jb * 16 + l
                    for k in range(HALF // 16):
                        rows[b][r, pl.ds(k * 16, 16)] = (
                            rows[b][r, pl.ds(k * 16, 16)] * wl
                        )

        for b in range(NBUF):
            _gather(b, b)

        @pl.loop(0, UPP, step=NBUF)
        def _round(i):
            for b in range(NBUF):
                u = i + b
                _gwait(b)
                _scale(u, b)
                pltpu.async_copy(rows[b], acc.at[dst_st.at[u]], ssem[b], add=True)
            for b in range(NBUF):
                pltpu.make_async_copy(rows[b], acc.at[dst_st.at[0]], ssem[b]).wait()
                _gather(i + NBUF + b, b)

        for b in range(NBUF):
            _gwait(b)

    plsc.subcore_barrier()

    for p in range(NPT // UNIT):
        pltpu.sync_copy(acc.at[pl.ds(s * NPT + p * UNIT, UNIT)], rows[0])
        pltpu.sync_copy(rows[0], out.at[c, pl.ds(s * NPT + p * UNIT, UNIT)])


def _mm1_body(dego_ref, degi_ref, h_ref, w1_ref, y_ref, rin_ref, rout_ref):
    d_out = dego_ref[0] + dego_ref[1]
    d_in = degi_ref[0] + degi_ref[1]
    r_out = lax.rsqrt(jnp.maximum(d_out, 1e-12))
    r_in = lax.rsqrt(jnp.maximum(d_in, 1e-12))
    rout_ref[...] = r_out
    rin_ref[...] = r_in
    x = h_ref[...] * r_out[:, None]
    y_ref[0] = jnp.dot(x, w1_ref[...], preferred_element_type=jnp.float32)


_MM1_ARGS = dict(
    grid=(NC, NB),
    in_specs=[
        pl.BlockSpec((NC, BT), lambda c, i: (0, i)),
        pl.BlockSpec((NC, BT), lambda c, i: (0, i)),
        pl.BlockSpec((BT, D), lambda c, i: (i, 0)),
        pl.BlockSpec((D, HALF), lambda c, i: (0, c)),
    ],
    out_specs=[
        pl.BlockSpec((1, BT, HALF), lambda c, i: (c, i, 0)),
        pl.BlockSpec((BT,), lambda c, i: (i,)),
        pl.BlockSpec((BT,), lambda c, i: (i,)),
    ],
    out_shape=[
        jax.ShapeDtypeStruct((NC, N_PAD, HALF), jnp.float32),
        jax.ShapeDtypeStruct((N_PAD,), jnp.float32),
        jax.ShapeDtypeStruct((N_PAD,), jnp.float32),
    ],
)
_mm1 = pl.pallas_call(_mm1_body, **_MM1_ARGS)


def _mm2_body(a_ref, rin_ref, rout_ref, b1_ref, w2_ref, y2_ref):
    rin = rin_ref[...][:, None]
    rout = rout_ref[...][:, None]
    h0 = jax.nn.relu(a_ref[0] * rin + b1_ref[0][None, :]) * rout
    h1 = jax.nn.relu(a_ref[1] * rin + b1_ref[1][None, :]) * rout
    y2_ref[0] = (
        jnp.dot(h0, w2_ref[0], preferred_element_type=jnp.float32)
        + jnp.dot(h1, w2_ref[1], preferred_element_type=jnp.float32)
    )


_MM2_ARGS = dict(
    grid=(NC, NB),
    in_specs=[
        pl.BlockSpec((NC, BT, HALF), lambda c, i: (0, i, 0)),
        pl.BlockSpec((BT,), lambda c, i: (i,)),
        pl.BlockSpec((BT,), lambda c, i: (i,)),
        pl.BlockSpec((NC, HALF), lambda c, i: (0, 0)),
        pl.BlockSpec((NC, HALF, HALF), lambda c, i: (0, 0, c)),
    ],
    out_specs=pl.BlockSpec((1, BT, HALF), lambda c, i: (c, i, 0)),
    out_shape=jax.ShapeDtypeStruct((NC, N_PAD, HALF), jnp.float32),
)
_mm2 = pl.pallas_call(_mm2_body, **_MM2_ARGS)


def _fin_body(a_ref, rin_ref, b2_ref, out_ref):
    rin = rin_ref[...][:, None]
    b2 = b2_ref[...]
    lo = a_ref[0] * rin + b2[:HALF][None, :]
    hi = a_ref[1] * rin + b2[HALF:][None, :]
    out_ref[...] = jnp.concatenate([lo, hi], axis=1)


_FIN_ARGS = dict(
    grid=(NB,),
    in_specs=[
        pl.BlockSpec((NC, BT, HALF), lambda i: (0, i, 0)),
        pl.BlockSpec((BT,), lambda i: (i,)),
        pl.BlockSpec((D,), lambda i: (0,)),
    ],
    out_specs=pl.BlockSpec((BT, D), lambda i: (i, 0)),
    out_shape=jax.ShapeDtypeStruct((N_PAD, D), jnp.float32),
)
_fin = pl.pallas_call(_fin_body, **_FIN_ARGS)


def kernel(h, edge_index, w, W1, b1, W2, b2):
    src = edge_index[0].astype(jnp.int32)
    dst = edge_index[1].astype(jnp.int32)
    w = w.astype(jnp.float32)
    pad = E_PAD - E
    srcR = jnp.concatenate([src, jnp.zeros((pad,), jnp.int32)]).reshape(UNITS, UNIT)
    dstR = jnp.concatenate([dst, jnp.zeros((pad,), jnp.int32)]).reshape(UNITS, UNIT)
    wR = jnp.concatenate([w, jnp.zeros((pad,), jnp.float32)]).reshape(UNITS, UNIT)

    deg_kernel = _make_deg_kernel()
    msg_kernel = _make_msg_kernel()
    h_p = jnp.zeros((N_PAD, D), jnp.float32).at[:N].set(h)
    dego, degi = deg_kernel(srcR, dstR, wR)
    y1, rin, rout = _mm1(dego, degi, h_p, W1)
    a1 = msg_kernel(y1.reshape(NC * N_PAD, HALF), srcR, dstR, wR)
    y2 = _mm2(a1, rin, rout, b1.reshape(NC, HALF), W2.reshape(NC, HALF, D))
    a2 = msg_kernel(y2.reshape(NC * N_PAD, HALF), srcR, dstR, wR)
    return _fin(a2, rin, b2)[:N]
